# baseline (speedup 1.0000x reference)
import functools

import jax
import jax.numpy as jnp
from jax import lax
from jax.experimental import pallas as pl
from jax.experimental.pallas import tpu as pltpu
from jax.experimental.pallas import tpu_sc as plsc

BATCH = 16384
DIM = 32
LANES = 16
IDX_CHUNK = 128


def _mf_dot_body(u_hbm, i_hbm, ue_hbm, ie_hbm, out_hbm,
                 uidx_v, iidx_v, urows_v, irows_v, out_v, sem_u, sem_i,
                 *, num_cores, bpw):
    wid = lax.axis_index("s") * num_cores + lax.axis_index("c")
    base = wid * bpw
    nchunk = bpw // IDX_CHUNK

    pltpu.sync_copy(u_hbm.at[pl.ds(wid * nchunk, nchunk)], uidx_v)
    pltpu.sync_copy(i_hbm.at[pl.ds(wid * nchunk, nchunk)], iidx_v)

    copies = []
    for k in range(nchunk):
        dst_u = urows_v.at[pl.ds(k * IDX_CHUNK, IDX_CHUNK)]
        dst_i = irows_v.at[pl.ds(k * IDX_CHUNK, IDX_CHUNK)]
        copies.append(pltpu.async_copy(ue_hbm.at[uidx_v.at[k]], dst_u, sem_u))
        copies.append(pltpu.async_copy(ie_hbm.at[iidx_v.at[k]], dst_i, sem_i))
    for c in copies:
        c.wait()

    row_iota = lax.iota(jnp.int32, LANES)
    col_idx = [jnp.full((LANES,), d, jnp.int32) for d in range(DIM)]

    def group_body(g, carry):
        ridx = g * LANES + row_iota
        acc = jnp.zeros((LANES,), jnp.float32)
        for d in range(DIM):
            uv = plsc.load_gather(urows_v, [ridx, col_idx[d]])
            iv = plsc.load_gather(irows_v, [ridx, col_idx[d]])
            acc = acc + uv * iv
        out_v[pl.ds(g * LANES, LANES)] = acc
        return carry

    lax.fori_loop(0, bpw // LANES, group_body, 0)
    pltpu.sync_copy(out_v, out_hbm.at[pl.ds(base, bpw)])


def kernel(u, i, user_embedding, item_embedding):
    info = plsc.get_sparse_core_info()
    nc, ns = info.num_cores, info.num_subcores
    nw = nc * ns
    bpw = BATCH // nw

    u2 = u.reshape(BATCH // IDX_CHUNK, IDX_CHUNK)
    i2 = i.reshape(BATCH // IDX_CHUNK, IDX_CHUNK)

    kfn = pl.kernel(
        functools.partial(_mf_dot_body, num_cores=nc, bpw=bpw),
        mesh=plsc.VectorSubcoreMesh(core_axis_name="c", subcore_axis_name="s"),
        out_type=jax.ShapeDtypeStruct((BATCH,), jnp.float32),
        scratch_types=[
            pltpu.VMEM((bpw // IDX_CHUNK, IDX_CHUNK), jnp.int32),
            pltpu.VMEM((bpw // IDX_CHUNK, IDX_CHUNK), jnp.int32),
            pltpu.VMEM((bpw, DIM), jnp.float32),
            pltpu.VMEM((bpw, DIM), jnp.float32),
            pltpu.VMEM((bpw,), jnp.float32),
            pltpu.SemaphoreType.DMA,
            pltpu.SemaphoreType.DMA,
        ],
        compiler_params=pltpu.CompilerParams(
            needs_layout_passes=False, use_tc_tiling_on_sc=False
        ),
    )
    return kfn(u2, i2, user_embedding, item_embedding)
